# baseline (speedup 1.0000x reference)
import functools

import jax
from jax import lax
from jax.experimental import pallas as pl
from jax.experimental.pallas import tpu as pltpu
from jax.experimental.pallas import tpu_sc as plsc

_INFO = plsc.get_sparse_core_info()
_NC, _NS = _INFO.num_cores, _INFO.num_subcores
_NW = _NC * _NS
_CHUNK_ROWS = 16
_NBUF = 3


def _make_sc_copy(sl, dim, dtype):
    rows_per_w = sl // _NW
    n_chunks = rows_per_w // _CHUNK_ROWS
    mesh = plsc.VectorSubcoreMesh(core_axis_name="c", subcore_axis_name="s")

    @functools.partial(
        pl.kernel,
        mesh=mesh,
        out_type=jax.ShapeDtypeStruct((sl, dim), dtype),
        scratch_types=[pltpu.SemaphoreType.DMA] * 4,
    )
    def sc_copy(tab, out, *sems):
        wid = lax.axis_index("s") * _NC + lax.axis_index("c")
        base = wid * rows_per_w
        step = rows_per_w // 4
        copies = [
            pltpu.async_copy(
                tab.at[pl.ds(base + j * step, step)],
                out.at[pl.ds(base + j * step, step)],
                sems[j],
            )
            for j in range(4)
        ]
        for c in copies:
            c.wait()

    return sc_copy


def kernel(x, emb_weight):
    sl = x.shape[1]
    dim = emb_weight.shape[1]
    return _make_sc_copy(sl, dim, emb_weight.dtype)(emb_weight[:sl])


# speedup vs baseline: 32.1686x; 32.1686x over previous
import functools

import jax
from jax import lax
from jax.experimental import pallas as pl
from jax.experimental.pallas import tpu as pltpu
from jax.experimental.pallas import tpu_sc as plsc

_INFO = plsc.get_sparse_core_info()
_NC, _NS = _INFO.num_cores, _INFO.num_subcores
_NW = _NC * _NS
_CHUNK_ROWS = 16
_NBUF = 3


def _make_sc_copy(sl, dim, dtype):
    rows_per_w = sl // _NW
    n_chunks = rows_per_w // _CHUNK_ROWS
    mesh = plsc.VectorSubcoreMesh(core_axis_name="c", subcore_axis_name="s")

    @functools.partial(
        pl.kernel,
        mesh=mesh,
        out_type=jax.ShapeDtypeStruct((sl, dim), dtype),
        scratch_types=(
            [pltpu.VMEM_SHARED((_NS, _CHUNK_ROWS, dim), dtype)] * _NBUF
            + [pltpu.SemaphoreType.DMA] * (2 * _NBUF)
        ),
    )
    def sc_copy(tab, out, *refs):
        sid = lax.axis_index("s")
        bufs = [r.at[sid] for r in refs[:_NBUF]]
        lsems = refs[_NBUF : 2 * _NBUF]
        ssems = refs[2 * _NBUF :]
        wid = sid * _NC + lax.axis_index("c")
        base = wid * rows_per_w

        def src(i):
            return tab.at[pl.ds(base + i * _CHUNK_ROWS, _CHUNK_ROWS)]

        def dst(i):
            return out.at[pl.ds(base + i * _CHUNK_ROWS, _CHUNK_ROWS)]

        loads = [None] * n_chunks
        stores = [None] * n_chunks
        for i in range(min(_NBUF, n_chunks)):
            loads[i] = pltpu.async_copy(src(i), bufs[i], lsems[i])
        for i in range(n_chunks):
            b = i % _NBUF
            loads[i].wait()
            stores[i] = pltpu.async_copy(bufs[b], dst(i), ssems[b])
            nxt = i + _NBUF
            if nxt < n_chunks:
                stores[i].wait()
                loads[nxt] = pltpu.async_copy(src(nxt), bufs[b], lsems[b])
        for i in range(max(0, n_chunks - _NBUF), n_chunks):
            if stores[i] is not None and i + _NBUF >= n_chunks:
                stores[i].wait()

    return sc_copy


def kernel(x, emb_weight):
    sl = x.shape[1]
    dim = emb_weight.shape[1]
    return _make_sc_copy(sl, dim, emb_weight.dtype)(emb_weight[:sl])
